# baseline (speedup 1.0000x reference)
import functools

import jax
import jax.numpy as jnp
from jax import lax
from jax.experimental import pallas as pl
from jax.experimental.pallas import tpu as pltpu
from jax.experimental.pallas import tpu_sc as plsc

B = 16384
D = 16
H = 32
V = 1000000

NC = 2
NS = 16
NW = NC * NS
BPW = B // NW
EPW = BPW * D
NCHUNK = EPW // 128

_W1_OFF = 64
_B1_OFF = _W1_OFF + H * D
_W2_OFF = _B1_OFF + H
_B2_OFF = _W2_OFF + H
_WLEN = 704

_MESH = plsc.VectorSubcoreMesh(core_axis_name="c", subcore_axis_name="s")


@functools.partial(
    pl.kernel,
    out_type=jax.ShapeDtypeStruct((B,), jnp.float32),
    mesh=_MESH,
    scratch_types=[
        pltpu.VMEM((NCHUNK, 128), jnp.int32),
        pltpu.VMEM((EPW,), jnp.float32),
        pltpu.VMEM((_WLEN,), jnp.float32),
        pltpu.VMEM((BPW,), jnp.float32),
        pltpu.SemaphoreType.DMA,
    ],
    compiler_params=pltpu.CompilerParams(needs_layout_passes=False),
)
def _sc_fused(idx_hbm, table_hbm, w_hbm, out_hbm, idx_v, ev_v, w_v, out_v, sem):
    c = lax.axis_index("c")
    s = lax.axis_index("s")
    wid = s * NC + c
    base = wid * BPW

    pltpu.sync_copy(idx_hbm.at[wid], idx_v)
    pltpu.sync_copy(w_hbm, w_v)

    copies = [
        pltpu.make_async_copy(
            table_hbm.at[idx_v.at[k]], ev_v.at[pl.ds(k * 128, 128)], sem)
        for k in range(NCHUNK)
    ]
    for cp in copies:
        cp.start()
    for cp in copies:
        cp.wait()

    def _wsplat(off):
        return plsc.load_gather(w_v, [jnp.full((16,), off, jnp.int32)])

    def grp_body(g, _):
        cols = [ev_v[pl.ds(g * 16 + d * BPW, 16)] for d in range(D)]
        acc = _wsplat(_B2_OFF)
        for j in range(H):
            h = _wsplat(_B1_OFF + j)
            for d in range(D):
                h = h + _wsplat(_W1_OFF + j * D + d) * cols[d]
            h = jnp.maximum(h, 0.0)
            acc = acc + _wsplat(_W2_OFF + j) * h
        out_v[pl.ds(g * 16, 16)] = acc
        return 0

    lax.fori_loop(0, BPW // 16, grp_body, 0)
    pltpu.sync_copy(out_v, out_hbm.at[pl.ds(base, BPW)])


def kernel(x, table, W1, b1, W2, b2):
    idx = x.astype(jnp.int32)
    flat = (jnp.arange(D, dtype=jnp.int32)[None, :, None] * V
            + idx.reshape(NW, 1, BPW))
    flat = flat.reshape(NW, NCHUNK, 128)
    table_flat = table.T.reshape(D * V)
    wflat = jnp.concatenate([
        jnp.zeros((_W1_OFF,), jnp.float32),
        W1.reshape(-1), b1, W2.reshape(-1), b2,
        jnp.zeros((_WLEN - _B2_OFF - 1,), jnp.float32),
    ])
    out = _sc_fused(flat, table_flat, wflat)
    return out.reshape(B, 1)


# speedup vs baseline: 6.8758x; 6.8758x over previous
import functools

import jax
import jax.numpy as jnp
from jax import lax
from jax.experimental import pallas as pl
from jax.experimental.pallas import tpu as pltpu
from jax.experimental.pallas import tpu_sc as plsc

B = 16384
D = 16
H = 32
V = 1000000

NC = 2
NS = 16
NW = NC * NS
BPW = B // NW
EPW = BPW * D
NCHUNK = EPW // 128

_W1_OFF = 64
_B1_OFF = _W1_OFF + H * D
_W2_OFF = _B1_OFF + H
_B2_OFF = _W2_OFF + H
_WLEN = 704

_MESH = plsc.VectorSubcoreMesh(core_axis_name="c", subcore_axis_name="s")

_VP = 1000064
_BLK = 8 * _VP
_W = 4096
_NSLAB = 488
_RMAIN = 244 * _W
_VTAIL0 = 999936
_NROW = 2 * _BLK // 1024 + 1


@functools.partial(
    pl.kernel,
    out_type=jax.ShapeDtypeStruct((_NROW, 8, 128), jnp.float32),
    mesh=_MESH,
    scratch_types=[
        pltpu.VMEM((8, _W), jnp.float32),
        pltpu.VMEM((8, 512), jnp.float32),
        pltpu.VMEM((8, 128), jnp.float32),
        pltpu.SemaphoreType.DMA,
    ],
    compiler_params=pltpu.CompilerParams(needs_layout_passes=False),
)
def _sc_relayout(tT_hbm, tail_hbm, flat_hbm, slab_v, rem_v, tail_v, sem):
    c = lax.axis_index("c")
    s = lax.axis_index("s")
    wid = s * NC + c

    def _emit_tiles(src_v, ntile, row0):
        copies = [
            pltpu.make_async_copy(
                src_v.at[:, pl.ds(t * 128, 128)], flat_hbm.at[row0 + t], sem)
            for t in range(ntile)
        ]
        for cp in copies:
            cp.start()
        for cp in copies:
            cp.wait()

    def slab_body(i, _):
        sl = wid + 32 * i
        h = sl % 2
        j = sl // 2
        pltpu.sync_copy(tT_hbm.at[pl.ds(h * 8, 8), pl.ds(j * _W, _W)], slab_v)
        _emit_tiles(slab_v, _W // 128, h * (_BLK // 1024) + j * (_W // 128))
        return 0

    nf = jnp.where(wid < _NSLAB % NW, (_NSLAB // NW) + 1, _NSLAB // NW)
    lax.fori_loop(0, nf, slab_body, 0)

    @pl.when(jnp.logical_or(wid == 8, wid == 9))
    def _rem():
        h = wid - 8
        pltpu.sync_copy(tT_hbm.at[pl.ds(h * 8, 8), pl.ds(_RMAIN, 512)], rem_v)
        _emit_tiles(rem_v, 4, h * (_BLK // 1024) + _RMAIN // 128)

    @pl.when(wid == 10)
    def _tail():
        pltpu.sync_copy(tail_hbm, tail_v)
        pltpu.sync_copy(tail_v, flat_hbm.at[_NROW - 1])


@functools.partial(
    pl.kernel,
    out_type=jax.ShapeDtypeStruct((B,), jnp.float32),
    mesh=_MESH,
    scratch_types=[
        pltpu.VMEM((NCHUNK, 128), jnp.int32),
        pltpu.VMEM((EPW,), jnp.float32),
        pltpu.VMEM((_WLEN,), jnp.float32),
        pltpu.VMEM((BPW,), jnp.float32),
        pltpu.SemaphoreType.DMA,
    ],
    compiler_params=pltpu.CompilerParams(needs_layout_passes=False),
)
def _sc_fused(idx_hbm, table_hbm, w_hbm, out_hbm, idx_v, ev_v, w_v, out_v, sem):
    c = lax.axis_index("c")
    s = lax.axis_index("s")
    wid = s * NC + c
    base = wid * BPW

    pltpu.sync_copy(idx_hbm.at[wid], idx_v)
    pltpu.sync_copy(w_hbm, w_v)

    copies = [
        pltpu.make_async_copy(
            table_hbm.at[idx_v.at[k]], ev_v.at[pl.ds(k * 128, 128)], sem)
        for k in range(NCHUNK)
    ]
    for cp in copies:
        cp.start()
    for cp in copies:
        cp.wait()

    def _wsplat(off):
        return plsc.load_gather(w_v, [jnp.full((16,), off, jnp.int32)])

    def grp_body(g, _):
        cols = [ev_v[pl.ds(g * 16 + d * BPW, 16)] for d in range(D)]
        acc = _wsplat(_B2_OFF)
        for j in range(H):
            h = _wsplat(_B1_OFF + j)
            for d in range(D):
                h = h + _wsplat(_W1_OFF + j * D + d) * cols[d]
            h = jnp.maximum(h, 0.0)
            acc = acc + _wsplat(_W2_OFF + j) * h
        out_v[pl.ds(g * 16, 16)] = acc
        return 0

    lax.fori_loop(0, BPW // 16, grp_body, 0)
    pltpu.sync_copy(out_v, out_hbm.at[pl.ds(base, BPW)])


def kernel(x, table, W1, b1, W2, b2):
    idx = x.astype(jnp.int32)
    dd = jnp.arange(D, dtype=jnp.int32)[None, :, None]
    rr = idx.reshape(NW, 1, BPW)
    flat = jnp.where(
        rr < _VTAIL0,
        (dd >> 3) * _BLK + (dd & 7) * 128 + (rr >> 7) * 1024 + (rr & 127),
        2 * _BLK + dd * 64 + (rr - _VTAIL0))
    flat = flat.reshape(NW, NCHUNK, 128)
    tail = table[_VTAIL0:, :].T.reshape(8, 128)
    table_flat = _sc_relayout(table.T, tail).reshape(_NROW * 1024)
    wflat = jnp.concatenate([
        jnp.zeros((_W1_OFF,), jnp.float32),
        W1.reshape(-1), b1, W2.reshape(-1), b2,
        jnp.zeros((_WLEN - _B2_OFF - 1,), jnp.float32),
    ])
    out = _sc_fused(flat, table_flat, wflat)
    return out.reshape(B, 1)


# speedup vs baseline: 12.9157x; 1.8784x over previous
import functools

import jax
import jax.numpy as jnp
from jax import lax
from jax.experimental import pallas as pl
from jax.experimental.pallas import tpu as pltpu
from jax.experimental.pallas import tpu_sc as plsc

B = 16384
D = 16
H = 32
V = 1000000

NC = 2
NS = 16
NW = NC * NS
BPW = B // NW
EPW = BPW * D
NCHUNK = EPW // 128

_W1_OFF = 64
_B1_OFF = _W1_OFF + H * D
_W2_OFF = _B1_OFF + H
_B2_OFF = _W2_OFF + H
_WLEN = 704

_MESH = plsc.VectorSubcoreMesh(core_axis_name="c", subcore_axis_name="s")

_VP = 1000064
_BLK = 8 * _VP
_W = 4096
_NSLAB = 488
_RMAIN = 244 * _W
_VTAIL0 = 999936
_NROW = 2 * _BLK // 1024 + 1


@functools.partial(
    pl.kernel,
    out_type=jax.ShapeDtypeStruct((_NROW, 8, 128), jnp.float32),
    mesh=_MESH,
    scratch_types=[
        pltpu.VMEM((8, _W), jnp.float32),
        pltpu.VMEM((8, 512), jnp.float32),
        pltpu.VMEM((8, 128), jnp.float32),
        pltpu.SemaphoreType.DMA,
    ],
    compiler_params=pltpu.CompilerParams(needs_layout_passes=False),
)
def _sc_relayout(tT_hbm, tail_hbm, flat_hbm, slab_v, rem_v, tail_v, sem):
    c = lax.axis_index("c")
    s = lax.axis_index("s")
    wid = s * NC + c

    def _emit_tiles(src_v, ntile, row0):
        copies = [
            pltpu.make_async_copy(
                src_v.at[:, pl.ds(t * 128, 128)], flat_hbm.at[row0 + t], sem)
            for t in range(ntile)
        ]
        for cp in copies:
            cp.start()
        for cp in copies:
            cp.wait()

    def slab_body(i, _):
        sl = wid + 32 * i
        h = sl % 2
        j = sl // 2
        pltpu.sync_copy(tT_hbm.at[pl.ds(h * 8, 8), pl.ds(j * _W, _W)], slab_v)
        _emit_tiles(slab_v, _W // 128, h * (_BLK // 1024) + j * (_W // 128))
        return 0

    nf = jnp.where(wid < _NSLAB % NW, (_NSLAB // NW) + 1, _NSLAB // NW)
    lax.fori_loop(0, nf, slab_body, 0)

    @pl.when(jnp.logical_or(wid == 8, wid == 9))
    def _rem():
        h = wid - 8
        pltpu.sync_copy(tT_hbm.at[pl.ds(h * 8, 8), pl.ds(_RMAIN, 512)], rem_v)
        _emit_tiles(rem_v, 4, h * (_BLK // 1024) + _RMAIN // 128)

    @pl.when(wid == 10)
    def _tail():
        pltpu.sync_copy(tail_hbm, tail_v)
        pltpu.sync_copy(tail_v, flat_hbm.at[_NROW - 1])


@functools.partial(
    pl.kernel,
    out_type=jax.ShapeDtypeStruct((B,), jnp.float32),
    mesh=_MESH,
    scratch_types=[
        pltpu.VMEM((NCHUNK, 128), jnp.int32),
        pltpu.VMEM((EPW,), jnp.float32),
        pltpu.VMEM((_WLEN,), jnp.float32),
        pltpu.VMEM((BPW,), jnp.float32),
        pltpu.SemaphoreType.DMA,
    ],
    compiler_params=pltpu.CompilerParams(needs_layout_passes=False),
)
def _sc_fused(idx_hbm, table_hbm, w_hbm, out_hbm, idx_v, ev_v, w_v, out_v, sem):
    c = lax.axis_index("c")
    s = lax.axis_index("s")
    wid = s * NC + c
    base = wid * BPW

    pltpu.sync_copy(idx_hbm.at[wid], idx_v)
    pltpu.sync_copy(w_hbm, w_v)

    copies = [
        pltpu.make_async_copy(
            table_hbm.at[idx_v.at[k]], ev_v.at[pl.ds(k * 128, 128)], sem)
        for k in range(NCHUNK)
    ]
    for cp in copies:
        cp.start()
    for cp in copies:
        cp.wait()

    def _wsplat(off):
        return plsc.load_gather(w_v, [jnp.full((16,), off, jnp.int32)])

    def grp_body(g, _):
        c0 = [ev_v[pl.ds(g * 32 + d * BPW, 16)] for d in range(D)]
        c1 = [ev_v[pl.ds(g * 32 + 16 + d * BPW, 16)] for d in range(D)]
        a0 = _wsplat(_B2_OFF)
        a1 = a0
        for j in range(H):
            h0 = _wsplat(_B1_OFF + j)
            h1 = h0
            for d in range(D):
                w = _wsplat(_W1_OFF + j * D + d)
                h0 = h0 + w * c0[d]
                h1 = h1 + w * c1[d]
            h0 = jnp.maximum(h0, 0.0)
            h1 = jnp.maximum(h1, 0.0)
            w2 = _wsplat(_W2_OFF + j)
            a0 = a0 + w2 * h0
            a1 = a1 + w2 * h1
        out_v[pl.ds(g * 32, 16)] = a0
        out_v[pl.ds(g * 32 + 16, 16)] = a1
        return 0

    lax.fori_loop(0, BPW // 32, grp_body, 0)
    pltpu.sync_copy(out_v, out_hbm.at[pl.ds(base, BPW)])


def kernel(x, table, W1, b1, W2, b2):
    idx = x.astype(jnp.int32)
    dd = jnp.arange(D, dtype=jnp.int32)[None, :, None]
    rr = idx.reshape(NW, 1, BPW)
    flat = jnp.where(
        rr < _VTAIL0,
        (dd >> 3) * _BLK + (dd & 7) * 128 + (rr >> 7) * 1024 + (rr & 127),
        2 * _BLK + dd * 64 + (rr - _VTAIL0))
    flat = flat.reshape(NW, NCHUNK, 128)
    tail = table[_VTAIL0:, :].T.reshape(8, 128)
    table_flat = _sc_relayout(table.T, tail).reshape(_NROW * 1024)
    wflat = jnp.concatenate([
        jnp.zeros((_W1_OFF,), jnp.float32),
        W1.reshape(-1), b1, W2.reshape(-1), b2,
        jnp.zeros((_WLEN - _B2_OFF - 1,), jnp.float32),
    ])
    out = _sc_fused(flat, table_flat, wflat)
    return out.reshape(B, 1)
